# Initial kernel scaffold; baseline (speedup 1.0000x reference)
import jax
import jax.numpy as jnp
from jax.experimental import pallas as pl


def kernel(x, edge_index, edge_attr, memory, batch_id, W_l, b_l, W_r, b_r, W_e, att, bias, ln_w, ln_b):
    raise NotImplementedError("write your pallas kernel here")


# speedup vs baseline: 2.1396x; 2.1396x over previous
import functools

import jax
import jax.numpy as jnp
from jax.experimental import pallas as pl
from jax.experimental.pallas import tpu as pltpu

N = 10000
E = 320000
D = 128
G = 8
EPS_LN = 1e-5

BN = 2000
BE = 2000


def _proj_body(x_ref, wl_ref, bl_ref, wr_ref, br_ref, xl_ref, xr_ref):
    x = x_ref[...]
    xl_ref[...] = jnp.dot(x, wl_ref[...], preferred_element_type=jnp.float32) + bl_ref[...]
    xr_ref[...] = jnp.dot(x, wr_ref[...], preferred_element_type=jnp.float32) + br_ref[...]


def _proj(x, W_l, b_l, W_r, b_r):
    nb = N // BN
    return pl.pallas_call(
        _proj_body,
        grid=(nb,),
        in_specs=[
            pl.BlockSpec((BN, D), lambda i: (i, 0)),
            pl.BlockSpec((D, D), lambda i: (0, 0)),
            pl.BlockSpec((1, D), lambda i: (0, 0)),
            pl.BlockSpec((D, D), lambda i: (0, 0)),
            pl.BlockSpec((1, D), lambda i: (0, 0)),
        ],
        out_specs=[
            pl.BlockSpec((BN, D), lambda i: (i, 0)),
            pl.BlockSpec((BN, D), lambda i: (i, 0)),
        ],
        out_shape=[
            jax.ShapeDtypeStruct((N, D), jnp.float32),
            jax.ShapeDtypeStruct((N, D), jnp.float32),
        ],
    )(x, W_l, b_l.reshape(1, D), W_r, b_r.reshape(1, D))


def _edge_body(ea_ref, gl_ref, gr_ref, we_ref, att_ref, ex_ref, msg_ref):
    ef = jnp.dot(ea_ref[...], we_ref[...], preferred_element_type=jnp.float32)
    m = gl_ref[...] + gr_ref[...] + ef
    m = jnp.where(m > 0, m, 0.2 * m)
    logits = jnp.dot(m, att_ref[...].reshape(D, 1), preferred_element_type=jnp.float32)
    ex = jnp.exp(logits)
    ex_ref[...] = ex
    msg_ref[...] = gl_ref[...] * ex


def _edge(edge_attr, g_l, g_r, W_e, att):
    nb = E // BE
    return pl.pallas_call(
        _edge_body,
        grid=(nb,),
        in_specs=[
            pl.BlockSpec((BE, D), lambda i: (i, 0)),
            pl.BlockSpec((BE, D), lambda i: (i, 0)),
            pl.BlockSpec((BE, D), lambda i: (i, 0)),
            pl.BlockSpec((D, D), lambda i: (0, 0)),
            pl.BlockSpec((1, D), lambda i: (0, 0)),
        ],
        out_specs=[
            pl.BlockSpec((BE, 1), lambda i: (i, 0)),
            pl.BlockSpec((BE, D), lambda i: (i, 0)),
        ],
        out_shape=[
            jax.ShapeDtypeStruct((E, 1), jnp.float32),
            jax.ShapeDtypeStruct((E, D), jnp.float32),
        ],
    )(edge_attr, g_l, g_r, W_e, att.reshape(1, D))


def _ln_body(x_ref, agg_ref, den_ref, bid_ref, bias_ref, lnw_ref, lnb_ref, out_ref):
    h = x_ref[...] + agg_ref[...] / (den_ref[...] + 1e-16) + bias_ref[...]
    bid = bid_ref[...]
    mean_pn = jnp.zeros((N, 1), jnp.float32)
    for g in range(G):
        mask = (bid == g)
        cnt = jnp.sum(mask.astype(jnp.float32))
        norm = jnp.maximum(cnt, 1.0) * D
        s = jnp.sum(jnp.where(mask, jnp.sum(h, axis=1, keepdims=True), 0.0))
        mean_pn = jnp.where(mask, s / norm, mean_pn)
    hc = h - mean_pn
    rstd_pn = jnp.zeros((N, 1), jnp.float32)
    for g in range(G):
        mask = (bid == g)
        cnt = jnp.sum(mask.astype(jnp.float32))
        norm = jnp.maximum(cnt, 1.0) * D
        v = jnp.sum(jnp.where(mask, jnp.sum(hc * hc, axis=1, keepdims=True), 0.0)) / norm
        rstd_pn = jnp.where(mask, jax.lax.rsqrt(v + EPS_LN), rstd_pn)
    out_ref[...] = hc * rstd_pn * lnw_ref[...] + lnb_ref[...]


def _lnorm(x, agg, denom, batch_id, bias, ln_w, ln_b):
    return pl.pallas_call(
        _ln_body,
        out_shape=jax.ShapeDtypeStruct((N, D), jnp.float32),
    )(x, agg, denom.reshape(N, 1), batch_id.reshape(N, 1),
      bias.reshape(1, D), ln_w.reshape(1, D), ln_b.reshape(1, D))


def kernel(x, edge_index, edge_attr, memory, batch_id, W_l, b_l, W_r, b_r,
           W_e, att, bias, ln_w, ln_b):
    src = edge_index[0]
    dst = edge_index[1]
    x_l, x_r = _proj(x, W_l, b_l, W_r, b_r)
    g_l = x_l[src]
    g_r = x_r[dst]
    ex, msg = _edge(edge_attr, g_l, g_r, W_e, att)
    ex = ex[:, 0]
    denom = jax.ops.segment_sum(ex, dst, num_segments=N)
    agg = jax.ops.segment_sum(msg, dst, num_segments=N)
    out = _lnorm(x, agg, denom, batch_id, bias, ln_w, ln_b)
    return (out, edge_attr)
